# baseline (speedup 1.0000x reference)
import math

import numpy as np
import jax
import jax.numpy as jnp
from jax.experimental import pallas as pl

H = 8
NEG = -3.0e38

_E_np = np.zeros((1024, 64), np.float32)
for _s in range(8):
    for _h in range(8):
        _E_np[_s * 128 + 16 * _h, _h * 8 + _s] = 1.0


def _qkv_kernel(x_ref, wq_ref, bq_ref, wk_ref, bk_ref, wv_ref, bv_ref,
                q_ref, k_ref, v_ref):
    x = x_ref[...]
    dn = (((1,), (0,)), ((), ()))
    q_ref[...] = jax.lax.dot_general(x, wq_ref[...], dn,
                                     preferred_element_type=jnp.float32) + bq_ref[...]
    k_ref[...] = jax.lax.dot_general(x, wk_ref[...], dn,
                                     preferred_element_type=jnp.float32) + bk_ref[...]
    v_ref[...] = jax.lax.dot_general(x, wv_ref[...], dn,
                                     preferred_element_type=jnp.float32) + bv_ref[...]


def _roll_l(t, k):
    return jnp.concatenate([t[:, k:], t[:, :k]], axis=1)


def _m_kernel(q_ref, k_ref, e_ref, wp_ref, bp_ref, m_ref):
    Q = q_ref[0]
    K8 = k_ref[0]
    parts = []
    for s in range(8):
        t0 = Q * K8[s:s + 1, :]
        t1 = t0 + _roll_l(t0, 8)
        t2 = t1 + _roll_l(t1, 4)
        t3 = t2 + _roll_l(t2, 2)
        t4 = t3 + _roll_l(t3, 1)
        parts.append(t4.astype(jnp.bfloat16).astype(jnp.float32))
    S = jnp.concatenate(parts, axis=1)
    dn = (((1,), (0,)), ((), ()))
    Tm = jax.lax.dot_general(S, e_ref[...], dn,
                             preferred_element_type=jnp.float32)
    wp = wp_ref[...]
    cols = []
    for h in range(8):
        cols.append(jax.lax.dot_general(Tm[:, 8 * h:8 * h + 8], wp, dn,
                                        preferred_element_type=jnp.float32))
    m_ref[0] = jnp.concatenate(cols, axis=1) + bp_ref[0]


def _argmax_steps(work, iota_row, steps, payload=None):
    picks = []
    for _ in range(steps):
        cm = jnp.max(work, axis=1, keepdims=True)
        ismax = work == cm
        pos = jnp.min(jnp.where(ismax, iota_row, 1.0e9), axis=1, keepdims=True)
        sel = iota_row == pos
        if payload is None:
            picks.append(pos)
        else:
            picks.append(jnp.sum(jnp.where(sel, payload, 0.0), axis=1, keepdims=True))
        work = jnp.where(sel, NEG, work)
    return jnp.concatenate(picks, axis=1)


def _make_sel_kernel(N, Tn, SN, mediam):
    NT = N - SN - mediam

    def sel_kernel(m_ref, mt_ref, g1_ref, g2_ref, ms_ref):
        iota_l = jax.lax.broadcasted_iota(jnp.int32, (N, N), 1).astype(jnp.float32)
        iota_c = jax.lax.broadcasted_iota(jnp.int32, (N, N), 0).astype(jnp.float32)
        iota_n = iota_l[0:1, :]
        for t in range(Tn):
            mrow = m_ref[0, t:t + 1, :]
            mcol = mt_ref[0, :, t:t + 1]
            lt = (mrow < mcol).astype(jnp.float32)
            eq = (mrow == mcol).astype(jnp.float32)
            jlt = (iota_l < iota_c).astype(jnp.float32)
            rank = (jnp.sum(lt, axis=1, keepdims=True)
                    + jnp.sum(eq * jlt, axis=1, keepdims=True))
            oh = (rank == iota_l).astype(jnp.float32)
            val_row = jnp.sum(oh * mcol, axis=0, keepdims=True)
            idx_row = jnp.sum(oh * iota_c, axis=0, keepdims=True)

            def region(vals, g):
                c = jnp.maximum(vals, 0.0)
                cmax = jnp.max(c, axis=1, keepdims=True)
                e = jnp.exp(c - cmax)
                ssum = jnp.sum(e, axis=1, keepdims=True)
                p = e / ssum
                return jnp.log(p + 1e-20) + g

            l_top = region(val_row[:, mediam:N - SN], g1_ref[0, t:t + 1, :])
            l_btm = region(val_row[:, :mediam], g2_ref[0, t:t + 1, :])

            top16 = _argmax_steps(mrow, iota_n, SN)
            rt = _argmax_steps(l_top, iota_n[:, :NT], SN // 2,
                               payload=idx_row[:, mediam:N - SN])
            rb = _argmax_steps(l_btm, iota_n[:, :mediam], SN // 2,
                               payload=idx_row[:, :mediam])
            ms_ref[0, t:t + 1, :] = jnp.concatenate([top16, rt, rb], axis=1)

    return sel_kernel


def _make_attn_kernel(N, S):
    def attn_kernel(q_ref, k_ref, v_ref, mst_ref,
                    wo_ref, bo_ref, wsk_ref, bsk_ref, wsv_ref, bsv_ref,
                    lnw_ref, lnb_ref, w1_ref, b1_ref, w2_ref, b2_ref,
                    out_ref, ap_ref, mp_ref):
        Q = q_ref[0]
        K = k_ref[0]
        V = v_ref[0]
        iota_n = jax.lax.broadcasted_iota(jnp.int32, (S, N), 1).astype(jnp.float32)
        dn = (((1,), (0,)), ((), ()))
        dnT = (((1,), (1,)), ((), ()))

        def softmax_l(x):
            cmax = jnp.max(x, axis=1, keepdims=True)
            e = jnp.exp(x - cmax)
            return e / jnp.sum(e, axis=1, keepdims=True)

        q_reds, k_reds, vals = [], [], []
        for h in range(H):
            mcol = mst_ref[0, :, h:h + 1]
            G = (mcol == iota_n).astype(jnp.float32)
            Qh = Q[:, 16 * h:16 * h + 16]
            Kh = K[:, 16 * h:16 * h + 16]
            Vh = V[:, 16 * h:16 * h + 16]
            q_red = jax.lax.dot_general(G, Qh, dn,
                                        preferred_element_type=jnp.float32)
            k_red = jax.lax.dot_general(G, Kh, dn,
                                        preferred_element_type=jnp.float32)
            qk = jax.lax.dot_general(q_red, Kh, dnT,
                                     preferred_element_type=jnp.float32) / 4.0
            attn = softmax_l(qk)
            vals.append(jax.lax.dot_general(attn, Vh, dn,
                                            preferred_element_type=jnp.float32))
            q_reds.append(q_red)
            k_reds.append(k_red)

        value_m = jnp.concatenate(vals, axis=1)
        projector = jnp.concatenate(q_reds, axis=1)
        k_red_m = jnp.concatenate(k_reds, axis=1)
        mp_ref[0] = value_m

        apk = jax.lax.dot_general(Q, k_red_m, dnT,
                                  preferred_element_type=jnp.float32) / 4.0
        ap_ref[0] = softmax_l(apk)

        sK = jax.lax.dot_general(projector, wsk_ref[...], dn,
                                 preferred_element_type=jnp.float32) + bsk_ref[...]
        sV = jax.lax.dot_general(value_m, wsv_ref[...], dn,
                                 preferred_element_type=jnp.float32) + bsv_ref[...]
        v2h = []
        for h in range(H):
            sqk = jax.lax.dot_general(Q[:, 16 * h:16 * h + 16],
                                      sK[:, 16 * h:16 * h + 16], dnT,
                                      preferred_element_type=jnp.float32) / 4.0
            a = softmax_l(sqk)
            vh = jax.lax.dot_general(a, sV[:, 16 * h:16 * h + 16], dn,
                                     preferred_element_type=jnp.float32)
            v2h.append(vh.astype(jnp.bfloat16).astype(jnp.float32))
        v2 = jnp.concatenate(v2h, axis=1)
        v2 = jax.lax.dot_general(v2, wo_ref[...], dn,
                                 preferred_element_type=jnp.float32) + bo_ref[...]
        mu = jnp.mean(v2, axis=1, keepdims=True)
        var = jnp.mean((v2 - mu) ** 2, axis=1, keepdims=True)
        v2 = (v2 - mu) / jnp.sqrt(var + 1e-5) * lnw_ref[...] + lnb_ref[...]
        hdd = jnp.maximum(jax.lax.dot_general(v2, w1_ref[...], dn,
                                              preferred_element_type=jnp.float32)
                          + b1_ref[...], 0.0)
        ffo = jax.lax.dot_general(hdd, w2_ref[...], dn,
                                  preferred_element_type=jnp.float32) + b2_ref[...]
        res = v2 + ffo
        mu2 = jnp.mean(res, axis=1, keepdims=True)
        var2 = jnp.mean((res - mu2) ** 2, axis=1, keepdims=True)
        out_ref[0] = (res - mu2) / jnp.sqrt(var2 + 1e-5)

    return attn_kernel


def kernel(x, Wq, bq, Wk, bk, Wv, bv, Wo, bo, Wsk, bsk, Wsv, bsv, Wproj, bproj,
           ln_w, ln_b, Wff1, bff1, Wff2, bff2, statica):
    B, T, N, D = x.shape
    BT = B * T
    SN = int(2 * math.log(N, 2))
    mediam = int((N - SN) / 2)
    NT = N - SN - mediam
    S = 2 * SN
    f32 = jnp.float32

    x2d = x.reshape(BT * N, D)
    RB = 800
    Gq = -(-x2d.shape[0] // RB)
    Q2, K2, V2 = pl.pallas_call(
        _qkv_kernel,
        grid=(Gq,),
        in_specs=[pl.BlockSpec((RB, D), lambda i: (i, 0))] +
                 [pl.BlockSpec((D, D), lambda i: (0, 0)),
                  pl.BlockSpec((1, D), lambda i: (0, 0))] * 3,
        out_specs=[pl.BlockSpec((RB, D), lambda i: (i, 0))] * 3,
        out_shape=[jax.ShapeDtypeStruct((x2d.shape[0], D), f32)] * 3,
    )(x2d, Wq, bq.reshape(1, D), Wk, bk.reshape(1, D), Wv, bv.reshape(1, D))

    Q3 = Q2.reshape(BT, N, D)
    K3 = K2.reshape(BT, N, D)
    V3 = V2.reshape(BT, N, D)

    E = jnp.asarray(_E_np)
    M = pl.pallas_call(
        _m_kernel,
        grid=(BT,),
        in_specs=[pl.BlockSpec((1, N, D), lambda i: (i, 0, 0)),
                  pl.BlockSpec((1, 8, D), lambda i: (i, 0, 0)),
                  pl.BlockSpec((1024, 64), lambda i: (0, 0)),
                  pl.BlockSpec((8, 1), lambda i: (0, 0)),
                  pl.BlockSpec((1,), lambda i: (0,))],
        out_specs=pl.BlockSpec((1, N, 8), lambda i: (i, 0, 0)),
        out_shape=jax.ShapeDtypeStruct((BT, N, 8), f32),
    )(Q3, K3, E, Wproj, bproj)

    Bh = B * H
    M4 = M.reshape(B, T, N, H).transpose(3, 0, 1, 2).reshape(Bh, T, N)
    M4t = M4.transpose(0, 2, 1)
    kk = jax.random.key(1234)
    k1, k2 = jax.random.split(kk)
    g1 = jax.random.gumbel(k1, (Bh * T, NT), dtype=f32).reshape(Bh, T, NT)
    g2 = jax.random.gumbel(k2, (Bh * T, mediam), dtype=f32).reshape(Bh, T, mediam)

    msel = pl.pallas_call(
        _make_sel_kernel(N, T, SN, mediam),
        grid=(Bh,),
        in_specs=[pl.BlockSpec((1, T, N), lambda i: (i, 0, 0)),
                  pl.BlockSpec((1, N, T), lambda i: (i, 0, 0)),
                  pl.BlockSpec((1, T, NT), lambda i: (i, 0, 0)),
                  pl.BlockSpec((1, T, mediam), lambda i: (i, 0, 0))],
        out_specs=pl.BlockSpec((1, T, S), lambda i: (i, 0, 0)),
        out_shape=jax.ShapeDtypeStruct((Bh, T, S), f32),
    )(M4, M4t, g1, g2)

    mst = msel.reshape(H, B, T, S).transpose(1, 2, 3, 0).reshape(BT, S, H)

    out4, ap4, mp4 = pl.pallas_call(
        _make_attn_kernel(N, S),
        grid=(BT,),
        in_specs=[pl.BlockSpec((1, N, D), lambda i: (i, 0, 0)),
                  pl.BlockSpec((1, N, D), lambda i: (i, 0, 0)),
                  pl.BlockSpec((1, N, D), lambda i: (i, 0, 0)),
                  pl.BlockSpec((1, S, H), lambda i: (i, 0, 0)),
                  pl.BlockSpec((D, D), lambda i: (0, 0)),
                  pl.BlockSpec((1, D), lambda i: (0, 0)),
                  pl.BlockSpec((D, D), lambda i: (0, 0)),
                  pl.BlockSpec((1, D), lambda i: (0, 0)),
                  pl.BlockSpec((D, D), lambda i: (0, 0)),
                  pl.BlockSpec((1, D), lambda i: (0, 0)),
                  pl.BlockSpec((1, D), lambda i: (0, 0)),
                  pl.BlockSpec((1, D), lambda i: (0, 0)),
                  pl.BlockSpec((D, D), lambda i: (0, 0)),
                  pl.BlockSpec((1, D), lambda i: (0, 0)),
                  pl.BlockSpec((D, D), lambda i: (0, 0)),
                  pl.BlockSpec((1, D), lambda i: (0, 0))],
        out_specs=[pl.BlockSpec((1, N, D), lambda i: (i, 0, 0)),
                   pl.BlockSpec((1, N, S), lambda i: (i, 0, 0)),
                   pl.BlockSpec((1, S, D), lambda i: (i, 0, 0))],
        out_shape=[jax.ShapeDtypeStruct((BT, N, D), f32),
                   jax.ShapeDtypeStruct((BT, N, S), f32),
                   jax.ShapeDtypeStruct((BT, S, D), f32)],
    )(Q3, K3, V3, mst,
      Wo, bo.reshape(1, D), Wsk, bsk.reshape(1, D), Wsv, bsv.reshape(1, D),
      ln_w.reshape(1, D), ln_b.reshape(1, D),
      Wff1, bff1.reshape(1, D), Wff2, bff2.reshape(1, D))

    out = out4.reshape(B, T, N, D)
    A_pro = ap4.reshape(B, T, N, S)
    M_pro = mp4.reshape(B, T, S, D)
    return out, A_pro, M_pro


# speedup vs baseline: 2.1284x; 2.1284x over previous
import math

import numpy as np
import jax
import jax.numpy as jnp
from jax.experimental import pallas as pl

H = 8
NEG = -3.0e38

_E_np = np.zeros((1024, 64), np.float32)
for _s in range(8):
    for _h in range(8):
        _E_np[_s * 128 + 16 * _h, _h * 8 + _s] = 1.0


def _qkv_kernel(x_ref, wq_ref, bq_ref, wk_ref, bk_ref, wv_ref, bv_ref,
                q_ref, k_ref, v_ref):
    x = x_ref[...]
    dn = (((1,), (0,)), ((), ()))
    q_ref[...] = jax.lax.dot_general(x, wq_ref[...], dn,
                                     preferred_element_type=jnp.float32) + bq_ref[...]
    k_ref[...] = jax.lax.dot_general(x, wk_ref[...], dn,
                                     preferred_element_type=jnp.float32) + bk_ref[...]
    v_ref[...] = jax.lax.dot_general(x, wv_ref[...], dn,
                                     preferred_element_type=jnp.float32) + bv_ref[...]


def _roll_l(t, k):
    return jnp.concatenate([t[:, k:], t[:, :k]], axis=1)


def _m_kernel(q_ref, k_ref, e_ref, wp_ref, bp_ref, m_ref):
    Q = q_ref[0]
    K8 = k_ref[0]
    parts = []
    for s in range(8):
        t0 = Q * K8[s:s + 1, :]
        t1 = t0 + _roll_l(t0, 8)
        t2 = t1 + _roll_l(t1, 4)
        t3 = t2 + _roll_l(t2, 2)
        t4 = t3 + _roll_l(t3, 1)
        parts.append(t4.astype(jnp.bfloat16).astype(jnp.float32))
    S = jnp.concatenate(parts, axis=1)
    dn = (((1,), (0,)), ((), ()))
    Tm = jax.lax.dot_general(S, e_ref[...], dn,
                             preferred_element_type=jnp.float32)
    wp = wp_ref[...]
    cols = []
    for h in range(8):
        cols.append(jax.lax.dot_general(Tm[:, 8 * h:8 * h + 8], wp, dn,
                                        preferred_element_type=jnp.float32))
    m_ref[0] = jnp.concatenate(cols, axis=1) + bp_ref[0]


def _argmax_steps(work, iota_row, steps, payload=None):
    picks = []
    for _ in range(steps):
        cm = jnp.max(work, axis=1, keepdims=True)
        ismax = work == cm
        pos = jnp.min(jnp.where(ismax, iota_row, 1.0e9), axis=1, keepdims=True)
        sel = iota_row == pos
        if payload is None:
            picks.append(pos)
        else:
            picks.append(jnp.sum(jnp.where(sel, payload, 0.0), axis=1, keepdims=True))
        work = jnp.where(sel, NEG, work)
    return jnp.concatenate(picks, axis=1)


def _make_sel_kernel(N, Tn, SN, mediam):
    NT = N - SN - mediam

    def sel_kernel(m_ref, g1_ref, g2_ref, ms_ref):
        iota_l = jax.lax.broadcasted_iota(jnp.int32, (N, N), 1).astype(jnp.float32)
        iota_c = jax.lax.broadcasted_iota(jnp.int32, (N, N), 0).astype(jnp.float32)
        iota_t = jax.lax.broadcasted_iota(jnp.int32, (Tn, N), 1).astype(jnp.float32)
        m = m_ref[0]
        mrow3 = m[:, None, :]
        mcol3 = m[:, :, None]
        lt = (mrow3 < mcol3).astype(jnp.float32)
        eq = (mrow3 == mcol3).astype(jnp.float32)
        jlt = (iota_l < iota_c).astype(jnp.float32)[None, :, :]
        rank = (jnp.sum(lt, axis=2, keepdims=True)
                + jnp.sum(eq * jlt, axis=2, keepdims=True))
        oh = (rank == iota_l[None, :, :]).astype(jnp.float32)
        val_row = jnp.sum(oh * mcol3, axis=1)
        idx_row = jnp.sum(oh * iota_c[None, :, :], axis=1)

        def region(vals, g):
            c = jnp.maximum(vals, 0.0)
            cmax = jnp.max(c, axis=1, keepdims=True)
            e = jnp.exp(c - cmax)
            ssum = jnp.sum(e, axis=1, keepdims=True)
            p = e / ssum
            return jnp.log(p + 1e-20) + g

        l_top = region(val_row[:, mediam:N - SN], g1_ref[0])
        l_btm = region(val_row[:, :mediam], g2_ref[0])

        top16 = _argmax_steps(m, iota_t, SN)
        rt = _argmax_steps(l_top, iota_t[:, :NT], SN // 2,
                           payload=idx_row[:, mediam:N - SN])
        rb = _argmax_steps(l_btm, iota_t[:, :mediam], SN // 2,
                           payload=idx_row[:, :mediam])
        ms_ref[0] = jnp.concatenate([top16, rt, rb], axis=1)

    return sel_kernel


def _make_attn_kernel(N, S):
    def attn_kernel(q_ref, k_ref, v_ref, mst_ref,
                    wo_ref, bo_ref, wsk_ref, bsk_ref, wsv_ref, bsv_ref,
                    lnw_ref, lnb_ref, w1_ref, b1_ref, w2_ref, b2_ref,
                    out_ref, ap_ref, mp_ref):
        Q = q_ref[0]
        K = k_ref[0]
        V = v_ref[0]
        iota_n = jax.lax.broadcasted_iota(jnp.int32, (S, N), 1).astype(jnp.float32)
        dn = (((1,), (0,)), ((), ()))
        dnT = (((1,), (1,)), ((), ()))

        def softmax_l(x):
            cmax = jnp.max(x, axis=1, keepdims=True)
            e = jnp.exp(x - cmax)
            return e / jnp.sum(e, axis=1, keepdims=True)

        q_reds, k_reds, vals = [], [], []
        for h in range(H):
            mcol = mst_ref[0, :, h:h + 1]
            G = (mcol == iota_n).astype(jnp.float32)
            Qh = Q[:, 16 * h:16 * h + 16]
            Kh = K[:, 16 * h:16 * h + 16]
            Vh = V[:, 16 * h:16 * h + 16]
            q_red = jax.lax.dot_general(G, Qh, dn,
                                        preferred_element_type=jnp.float32)
            k_red = jax.lax.dot_general(G, Kh, dn,
                                        preferred_element_type=jnp.float32)
            qk = jax.lax.dot_general(q_red, Kh, dnT,
                                     preferred_element_type=jnp.float32) / 4.0
            attn = softmax_l(qk)
            vals.append(jax.lax.dot_general(attn, Vh, dn,
                                            preferred_element_type=jnp.float32))
            q_reds.append(q_red)
            k_reds.append(k_red)

        value_m = jnp.concatenate(vals, axis=1)
        projector = jnp.concatenate(q_reds, axis=1)
        k_red_m = jnp.concatenate(k_reds, axis=1)
        mp_ref[0] = value_m

        apk = jax.lax.dot_general(Q, k_red_m, dnT,
                                  preferred_element_type=jnp.float32) / 4.0
        ap_ref[0] = softmax_l(apk)

        sK = jax.lax.dot_general(projector, wsk_ref[...], dn,
                                 preferred_element_type=jnp.float32) + bsk_ref[...]
        sV = jax.lax.dot_general(value_m, wsv_ref[...], dn,
                                 preferred_element_type=jnp.float32) + bsv_ref[...]
        v2h = []
        for h in range(H):
            sqk = jax.lax.dot_general(Q[:, 16 * h:16 * h + 16],
                                      sK[:, 16 * h:16 * h + 16], dnT,
                                      preferred_element_type=jnp.float32) / 4.0
            a = softmax_l(sqk)
            vh = jax.lax.dot_general(a, sV[:, 16 * h:16 * h + 16], dn,
                                     preferred_element_type=jnp.float32)
            v2h.append(vh.astype(jnp.bfloat16).astype(jnp.float32))
        v2 = jnp.concatenate(v2h, axis=1)
        v2 = jax.lax.dot_general(v2, wo_ref[...], dn,
                                 preferred_element_type=jnp.float32) + bo_ref[...]
        mu = jnp.mean(v2, axis=1, keepdims=True)
        var = jnp.mean((v2 - mu) ** 2, axis=1, keepdims=True)
        v2 = (v2 - mu) / jnp.sqrt(var + 1e-5) * lnw_ref[...] + lnb_ref[...]
        hdd = jnp.maximum(jax.lax.dot_general(v2, w1_ref[...], dn,
                                              preferred_element_type=jnp.float32)
                          + b1_ref[...], 0.0)
        ffo = jax.lax.dot_general(hdd, w2_ref[...], dn,
                                  preferred_element_type=jnp.float32) + b2_ref[...]
        res = v2 + ffo
        mu2 = jnp.mean(res, axis=1, keepdims=True)
        var2 = jnp.mean((res - mu2) ** 2, axis=1, keepdims=True)
        out_ref[0] = (res - mu2) / jnp.sqrt(var2 + 1e-5)

    return attn_kernel


def kernel(x, Wq, bq, Wk, bk, Wv, bv, Wo, bo, Wsk, bsk, Wsv, bsv, Wproj, bproj,
           ln_w, ln_b, Wff1, bff1, Wff2, bff2, statica):
    B, T, N, D = x.shape
    BT = B * T
    SN = int(2 * math.log(N, 2))
    mediam = int((N - SN) / 2)
    NT = N - SN - mediam
    S = 2 * SN
    f32 = jnp.float32

    x2d = x.reshape(BT * N, D)
    RB = 800
    Gq = -(-x2d.shape[0] // RB)
    Q2, K2, V2 = pl.pallas_call(
        _qkv_kernel,
        grid=(Gq,),
        in_specs=[pl.BlockSpec((RB, D), lambda i: (i, 0))] +
                 [pl.BlockSpec((D, D), lambda i: (0, 0)),
                  pl.BlockSpec((1, D), lambda i: (0, 0))] * 3,
        out_specs=[pl.BlockSpec((RB, D), lambda i: (i, 0))] * 3,
        out_shape=[jax.ShapeDtypeStruct((x2d.shape[0], D), f32)] * 3,
    )(x2d, Wq, bq.reshape(1, D), Wk, bk.reshape(1, D), Wv, bv.reshape(1, D))

    Q3 = Q2.reshape(BT, N, D)
    K3 = K2.reshape(BT, N, D)
    V3 = V2.reshape(BT, N, D)

    E = jnp.asarray(_E_np)
    M = pl.pallas_call(
        _m_kernel,
        grid=(BT,),
        in_specs=[pl.BlockSpec((1, N, D), lambda i: (i, 0, 0)),
                  pl.BlockSpec((1, 8, D), lambda i: (i, 0, 0)),
                  pl.BlockSpec((1024, 64), lambda i: (0, 0)),
                  pl.BlockSpec((8, 1), lambda i: (0, 0)),
                  pl.BlockSpec((1,), lambda i: (0,))],
        out_specs=pl.BlockSpec((1, N, 8), lambda i: (i, 0, 0)),
        out_shape=jax.ShapeDtypeStruct((BT, N, 8), f32),
    )(Q3, K3, E, Wproj, bproj)

    Bh = B * H
    M4 = M.reshape(B, T, N, H).transpose(3, 0, 1, 2).reshape(Bh, T, N)
    kk = jax.random.key(1234)
    k1, k2 = jax.random.split(kk)
    g1 = jax.random.gumbel(k1, (Bh * T, NT), dtype=f32).reshape(Bh, T, NT)
    g2 = jax.random.gumbel(k2, (Bh * T, mediam), dtype=f32).reshape(Bh, T, mediam)

    msel = pl.pallas_call(
        _make_sel_kernel(N, T, SN, mediam),
        grid=(Bh,),
        in_specs=[pl.BlockSpec((1, T, N), lambda i: (i, 0, 0)),
                  pl.BlockSpec((1, T, NT), lambda i: (i, 0, 0)),
                  pl.BlockSpec((1, T, mediam), lambda i: (i, 0, 0))],
        out_specs=pl.BlockSpec((1, T, S), lambda i: (i, 0, 0)),
        out_shape=jax.ShapeDtypeStruct((Bh, T, S), f32),
    )(M4, g1, g2)

    mst = msel.reshape(H, B, T, S).transpose(1, 2, 3, 0).reshape(BT, S, H)

    out4, ap4, mp4 = pl.pallas_call(
        _make_attn_kernel(N, S),
        grid=(BT,),
        in_specs=[pl.BlockSpec((1, N, D), lambda i: (i, 0, 0)),
                  pl.BlockSpec((1, N, D), lambda i: (i, 0, 0)),
                  pl.BlockSpec((1, N, D), lambda i: (i, 0, 0)),
                  pl.BlockSpec((1, S, H), lambda i: (i, 0, 0)),
                  pl.BlockSpec((D, D), lambda i: (0, 0)),
                  pl.BlockSpec((1, D), lambda i: (0, 0)),
                  pl.BlockSpec((D, D), lambda i: (0, 0)),
                  pl.BlockSpec((1, D), lambda i: (0, 0)),
                  pl.BlockSpec((D, D), lambda i: (0, 0)),
                  pl.BlockSpec((1, D), lambda i: (0, 0)),
                  pl.BlockSpec((1, D), lambda i: (0, 0)),
                  pl.BlockSpec((1, D), lambda i: (0, 0)),
                  pl.BlockSpec((D, D), lambda i: (0, 0)),
                  pl.BlockSpec((1, D), lambda i: (0, 0)),
                  pl.BlockSpec((D, D), lambda i: (0, 0)),
                  pl.BlockSpec((1, D), lambda i: (0, 0))],
        out_specs=[pl.BlockSpec((1, N, D), lambda i: (i, 0, 0)),
                   pl.BlockSpec((1, N, S), lambda i: (i, 0, 0)),
                   pl.BlockSpec((1, S, D), lambda i: (i, 0, 0))],
        out_shape=[jax.ShapeDtypeStruct((BT, N, D), f32),
                   jax.ShapeDtypeStruct((BT, N, S), f32),
                   jax.ShapeDtypeStruct((BT, S, D), f32)],
    )(Q3, K3, V3, mst,
      Wo, bo.reshape(1, D), Wsk, bsk.reshape(1, D), Wsv, bsv.reshape(1, D),
      ln_w.reshape(1, D), ln_b.reshape(1, D),
      Wff1, bff1.reshape(1, D), Wff2, bff2.reshape(1, D))

    out = out4.reshape(B, T, N, D)
    A_pro = ap4.reshape(B, T, N, S)
    M_pro = mp4.reshape(B, T, S, D)
    return out, A_pro, M_pro
